# Initial kernel scaffold; baseline (speedup 1.0000x reference)
import jax
import jax.numpy as jnp
from jax.experimental import pallas as pl


def kernel(ob_no, histogram, n):
    raise NotImplementedError("write your pallas kernel here")


# speedup vs baseline: 291.9422x; 291.9422x over previous
import functools

import jax
import jax.numpy as jnp
from jax import lax
from jax.experimental import pallas as pl
from jax.experimental.pallas import tpu as pltpu, tpu_sc as plsc

N = 4_194_304
M = 1_000_000

_info = plsc.get_sparse_core_info()
_NC, _NS = _info.num_cores, _info.num_subcores
_NW = _NC * _NS
_PER_W = N // _NW
_CHUNK = 16_384
_NCHUNKS = _PER_W // _CHUNK


_G = 16
_OB_ROWS = 512
_H_ROWS = 128
_LANE = 8192


def _prep_body(v_ref, ob_ref, h_ref, idx_ref, t2_ref):
    idx_ref[...] = ob_ref[...].astype(jnp.int32)
    t2_ref[...] = jnp.sqrt(v_ref[0] / (h_ref[...] + 1.0))


def _tc_prep(ob2d, h2d, v):
    return pl.pallas_call(
        _prep_body,
        grid=(_G,),
        in_specs=[
            pl.BlockSpec(memory_space=pltpu.SMEM),
            pl.BlockSpec((_OB_ROWS // _G, _LANE), lambda i: (i, 0)),
            pl.BlockSpec((_H_ROWS // _G, _LANE), lambda i: (i, 0)),
        ],
        out_specs=[
            pl.BlockSpec((_OB_ROWS // _G, _LANE), lambda i: (i, 0)),
            pl.BlockSpec((_H_ROWS // _G, _LANE), lambda i: (i, 0)),
        ],
        out_shape=[
            jax.ShapeDtypeStruct((_OB_ROWS, _LANE), jnp.int32),
            jax.ShapeDtypeStruct((_H_ROWS, _LANE), jnp.float32),
        ],
    )(v, ob2d, h2d)


_MP = _H_ROWS * _LANE
_SEG = _MP // _NS

_mesh = plsc.VectorSubcoreMesh(core_axis_name="c", subcore_axis_name="s")


@functools.partial(
    pl.kernel,
    out_type=jax.ShapeDtypeStruct((N,), jnp.float32),
    mesh=_mesh,
    scratch_types=[
        pltpu.VMEM((_CHUNK,), jnp.int32),
        pltpu.VMEM((_CHUNK,), jnp.float32),
        pltpu.VMEM_SHARED((_MP,), jnp.float32),
        pltpu.SemaphoreType.DMA,
    ],
)
def _sc_gather(t2_hbm, idx_hbm, out_hbm, idx_v, val_v, table_sh, sem):
    cid = lax.axis_index("c")
    sid = lax.axis_index("s")
    wid = sid * _NC + cid
    pltpu.sync_copy(
        t2_hbm.at[pl.ds(sid * _SEG, _SEG)], table_sh.at[pl.ds(sid * _SEG, _SEG)]
    )
    plsc.subcore_barrier()

    base = wid * _PER_W

    def chunk(i, carry):
        off = base + i * _CHUNK
        pltpu.sync_copy(idx_hbm.at[pl.ds(off, _CHUNK)], idx_v)
        pltpu.async_copy(table_sh.at[idx_v], val_v, sem).wait()
        pltpu.sync_copy(val_v, out_hbm.at[pl.ds(off, _CHUNK)])
        return carry

    lax.fori_loop(0, _NCHUNKS, chunk, 0)


def kernel(ob_no, histogram, n):
    n_new = jnp.asarray(n + ob_no.shape[0], jnp.float32)
    v = (2.0 * jnp.log(n_new)).reshape((1,))
    ob2d = ob_no.reshape(_OB_ROWS, _LANE)
    h2d = jnp.pad(histogram, (0, _MP - M)).reshape(_H_ROWS, _LANE)
    idx2d, t2_2d = _tc_prep(ob2d, h2d, v)
    out = _sc_gather(t2_2d.reshape(_MP), idx2d.reshape(N))
    return out


# speedup vs baseline: 576.8469x; 1.9759x over previous
import functools

import jax
import jax.numpy as jnp
from jax import lax
from jax.experimental import pallas as pl
from jax.experimental.pallas import tpu as pltpu, tpu_sc as plsc

N = 4_194_304
M = 1_000_000
_MP = 1_048_576

_info = plsc.get_sparse_core_info()
_NC, _NS = _info.num_cores, _info.num_subcores
_NW = _NC * _NS
_PER_W = N // _NW
_CHUNK = 8_192
_NCHUNKS = _PER_W // _CHUNK
_SEG = _MP // _NS


def _table_body(v_ref, h_ref, t2_ref):
    t2_ref[pl.ds(0, M)] = jnp.sqrt(v_ref[0] / (h_ref[...] + 1.0))


def _tc_table(hist, v):
    return pl.pallas_call(
        _table_body,
        in_specs=[
            pl.BlockSpec(memory_space=pltpu.SMEM),
            pl.BlockSpec(memory_space=pltpu.VMEM),
        ],
        out_specs=pl.BlockSpec(memory_space=pltpu.VMEM),
        out_shape=jax.ShapeDtypeStruct((_MP,), jnp.float32),
    )(v, hist)


_mesh = plsc.VectorSubcoreMesh(core_axis_name="c", subcore_axis_name="s")


@functools.partial(
    pl.kernel,
    out_type=jax.ShapeDtypeStruct((N,), jnp.float32),
    mesh=_mesh,
    scratch_types=[
        pltpu.VMEM((_CHUNK,), jnp.float32),
        pltpu.VMEM((_CHUNK,), jnp.float32),
        pltpu.VMEM((_CHUNK,), jnp.int32),
        pltpu.VMEM((_CHUNK,), jnp.int32),
        pltpu.VMEM((_CHUNK,), jnp.float32),
        pltpu.VMEM((_CHUNK,), jnp.float32),
        pltpu.VMEM_SHARED((_MP,), jnp.float32),
        pltpu.SemaphoreType.DMA,
        pltpu.SemaphoreType.DMA,
        pltpu.SemaphoreType.DMA,
        pltpu.SemaphoreType.DMA,
        pltpu.SemaphoreType.DMA,
    ],
)
def _sc_gather(t2_hbm, ob_hbm, out_hbm, ob0, ob1, idx0, idx1, val0, val1,
               table_sh, obs0, obs1, gsem, wbs0, wbs1):
    cid = lax.axis_index("c")
    sid = lax.axis_index("s")
    wid = sid * _NC + cid
    ob_v = (ob0, ob1)
    idx_v = (idx0, idx1)
    val_v = (val0, val1)
    obsems = (obs0, obs1)
    wbsems = (wbs0, wbs1)

    pltpu.sync_copy(
        t2_hbm.at[pl.ds(sid * _SEG, _SEG)], table_sh.at[pl.ds(sid * _SEG, _SEG)]
    )
    plsc.subcore_barrier()

    base = wid * _PER_W

    def start_ob(i):
        b = i & 1
        return pltpu.async_copy(
            ob_hbm.at[pl.ds(base + i * _CHUNK, _CHUNK)], ob_v[b], obsems[b]
        )

    def convert(i):
        b = i & 1

        @plsc.parallel_loop(0, _CHUNK, step=16, unroll=8)
        def _(j):
            s = pl.ds(j, 16)
            idx_v[b][s] = ob_v[b][s].astype(jnp.int32)

    def start_gather(i):
        b = i & 1
        return pltpu.async_copy(table_sh.at[idx_v[b]], val_v[b], gsem)

    def start_wb(i):
        b = i & 1
        return pltpu.async_copy(
            val_v[b], out_hbm.at[pl.ds(base + i * _CHUNK, _CHUNK)], wbsems[b]
        )

    ob_d = {0: start_ob(0)}
    g_d, wb_d = {}, {}
    ob_d[0].wait()
    convert(0)
    g_d[0] = start_gather(0)
    if _NCHUNKS > 1:
        ob_d[1] = start_ob(1)
    for i in range(_NCHUNKS):
        if i + 1 < _NCHUNKS:
            ob_d[i + 1].wait()
            convert(i + 1)
            if i + 2 < _NCHUNKS:
                ob_d[i + 2] = start_ob(i + 2)
        g_d[i].wait()
        wb_d[i] = start_wb(i)
        if i + 1 < _NCHUNKS:
            if i >= 1:
                wb_d[i - 1].wait()
            g_d[i + 1] = start_gather(i + 1)
    wb_d[_NCHUNKS - 2].wait()
    wb_d[_NCHUNKS - 1].wait()


def kernel(ob_no, histogram, n):
    n_new = jnp.asarray(n + ob_no.shape[0], jnp.float32)
    v = (2.0 * jnp.log(n_new)).reshape((1,))
    t2 = _tc_table(histogram, v)
    return _sc_gather(t2, ob_no)
